# baseline (speedup 1.0000x reference)
import functools

import jax
import jax.numpy as jnp
import numpy as np
from jax import lax
from jax.experimental import pallas as pl
from jax.experimental.pallas import tpu as pltpu
from jax.experimental.pallas import tpu_sc as plsc

VOCAB = 1000000
D = 64
HID = 128
NCLS = 2
B = 4096
L = 200

NW = 32
SPW = B // NW
IPW = SPW * L
WINDOW = 16

CW = 256
NFULL = VOCAB // CW
RAG = VOCAB - NFULL * CW
RAG_C0 = NFULL * CW
CPW = (NFULL + NW - 1) // NW

_mesh = plsc.VectorSubcoreMesh(core_axis_name="c", subcore_axis_name="s")


@functools.partial(
    pl.kernel,
    out_type=jax.ShapeDtypeStruct((VOCAB, 2 * D), jnp.float32),
    mesh=_mesh,
    scratch_types=[
        pltpu.VMEM((D, CW), jnp.float32),
        pltpu.VMEM((D, CW), jnp.float32),
        pltpu.VMEM((CW, 2 * D), jnp.float32),
        pltpu.VMEM((CW, 2 * D), jnp.float32),
        pltpu.SemaphoreType.DMA,
        pltpu.SemaphoreType.DMA,
        pltpu.SemaphoreType.DMA,
        pltpu.SemaphoreType.DMA,
    ],
    compiler_params=pltpu.CompilerParams(needs_layout_passes=False),
)
def _sc_format(embt_hbm, tail_hbm, embd_hbm, slab0, slab1, out0, out1,
               sin0, sin1, sout0, sout1):
    wid = lax.axis_index("s") * 2 + lax.axis_index("c")
    lane = lax.iota(jnp.int32, 16)
    slabs = (slab0, slab1)
    outs = (out0, out1)
    sins = (sin0, sin1)
    souts = (sout0, sout1)

    def in_desc(j, b):
        c0 = pl.multiple_of(j * CW, CW)
        return pltpu.make_async_copy(embt_hbm.at[:, pl.ds(c0, CW)],
                                     slabs[b], sins[b])

    def out_desc(j, b):
        c0 = pl.multiple_of(j * CW, CW)
        return pltpu.make_async_copy(outs[b], embd_hbm.at[pl.ds(c0, CW)],
                                     souts[b])

    for b in range(2):
        j = wid + b * NW

        @pl.when(j < NFULL)
        def _():
            in_desc(j, b).start()

    @pl.loop(0, CPW, step=2)
    def _chunks(k):
        for b in range(2):
            j = wid + (k + b) * NW

            @pl.when(j < NFULL)
            def _():
                in_desc(j, b).wait()

                @pl.when(k + b >= 2)
                def _():
                    out_desc(j, b).wait()

                @pl.loop(0, CW, unroll=8)
                def _row(r):
                    col = jnp.full((16,), r, jnp.int32)
                    for gi in range(D // 16):
                        v = plsc.load_gather(slabs[b], [lane + gi * 16, col])
                        outs[b][r, pl.ds(gi * 16, 16)] = v
                        outs[b][r, pl.ds(D + gi * 16, 16)] = v

                nj = j + 2 * NW

                @pl.when(nj < NFULL)
                def _():
                    in_desc(nj, b).start()

                out_desc(j, b).start()

    for b in range(2):
        last = wid + (CPW - 2 + b) * NW

        @pl.when(last < NFULL)
        def _():
            out_desc(last, b).wait()

    @pl.when(wid == 1)
    def _ragged():
        pltpu.sync_copy(tail_hbm, out0.at[pl.ds(0, RAG)])
        pltpu.sync_copy(out0.at[pl.ds(0, RAG)],
                        embd_hbm.at[pl.ds(RAG_C0, RAG)])


@functools.partial(
    pl.kernel,
    out_type=jax.ShapeDtypeStruct((B, 2 * D), jnp.float32),
    mesh=_mesh,
    scratch_types=[
        pltpu.VMEM((IPW,), jnp.int32),
        pltpu.VMEM((L, SPW), jnp.int32),
        pltpu.VMEM((SPW, 2 * D), jnp.float32),
        pltpu.SemaphoreType.DMA,
    ],
    compiler_params=pltpu.CompilerParams(needs_layout_passes=False),
)
def _sc_pool(x_hbm, embd_hbm, out_hbm, xb_v, idx_v, acc_v, sem):
    wid = lax.axis_index("s") * 2 + lax.axis_index("c")
    pltpu.sync_copy(x_hbm.at[pl.ds(wid * IPW, IPW)], xb_v)

    lane = lax.iota(jnp.int32, 16)

    @pl.loop(0, L, unroll=8)
    def _tr(r):
        base = lane * L + r
        for gi in range(SPW // 16):
            v = plsc.load_gather(xb_v, [base + (gi * 16 * L)])
            idx_v[r, pl.ds(gi * 16, 16)] = v

    pltpu.sync_copy(embd_hbm.at[idx_v.at[0]], acc_v)

    @pl.loop(0, L - 1)
    def _fire(i):
        pltpu.async_copy(embd_hbm.at[idx_v.at[i + 1]], acc_v, sem, add=True)

        @pl.when(i >= WINDOW - 1)
        def _():
            pltpu.make_async_copy(embd_hbm.at[idx_v.at[0]], acc_v, sem).wait()

    @pl.loop(0, WINDOW - 1)
    def _drain(_):
        pltpu.make_async_copy(embd_hbm.at[idx_v.at[0]], acc_v, sem).wait()

    pltpu.sync_copy(acc_v, out_hbm.at[pl.ds(wid * SPW, SPW)])


def _mlp_body(s_ref, w1_ref, b1_ref, w2_ref, b2_ref, o_ref):
    x = s_ref[...] * np.float32(1.0 / L)
    h = lax.dot_general(x, w1_ref[...], (((1,), (1,)), ((), ())),
                        preferred_element_type=jnp.float32)
    h = jnp.maximum(h + b1_ref[...], 0.0)
    o_ref[...] = lax.dot_general(h, w2_ref[...], (((1,), (1,)), ((), ())),
                                 preferred_element_type=jnp.float32) + b2_ref[...]


def _mlp(sums, W1, b1, W2, b2):
    return pl.pallas_call(
        _mlp_body,
        out_shape=jax.ShapeDtypeStruct((B, NCLS), jnp.float32),
    )(sums, W1, b1.reshape(1, HID), W2, b2.reshape(1, NCLS))


def kernel(x_in, emb, W1, b1, W2, b2):
    embt = jnp.transpose(emb)
    tail = emb[RAG_C0:]
    tail = jnp.concatenate([tail, tail], axis=1)
    embd = _sc_format(embt, tail)
    sums = _sc_pool(x_in.reshape(-1), embd)[:, :D]
    return _mlp(sums, W1, b1, W2, b2)


# speedup vs baseline: 1.6918x; 1.6918x over previous
import functools

import jax
import jax.numpy as jnp
import numpy as np
from jax import lax
from jax.experimental import pallas as pl
from jax.experimental.pallas import tpu as pltpu
from jax.experimental.pallas import tpu_sc as plsc

VOCAB = 1000000
D = 64
HID = 128
NCLS = 2
B = 4096
L = 200

NW = 32
SPW = B // NW
IPW = SPW * L
WINDOW = 16

CW = 256
NFULL = VOCAB // CW
RAG = VOCAB - NFULL * CW
RAG_C0 = NFULL * CW
CPW = (NFULL + NW - 1) // NW

_mesh = plsc.VectorSubcoreMesh(core_axis_name="c", subcore_axis_name="s")


@functools.partial(
    pl.kernel,
    out_type=jax.ShapeDtypeStruct((VOCAB, 2 * D), jnp.float32),
    mesh=_mesh,
    scratch_types=[
        pltpu.VMEM((D, CW), jnp.float32),
        pltpu.VMEM((D, CW), jnp.float32),
        pltpu.VMEM((CW, 2 * D), jnp.float32),
        pltpu.VMEM((CW, 2 * D), jnp.float32),
        pltpu.SemaphoreType.DMA,
        pltpu.SemaphoreType.DMA,
        pltpu.SemaphoreType.DMA,
        pltpu.SemaphoreType.DMA,
    ],
    compiler_params=pltpu.CompilerParams(needs_layout_passes=False),
)
def _sc_format(embt_hbm, tail_hbm, embd_hbm, slab0, slab1, out0, out1,
               sin0, sin1, sout0, sout1):
    wid = lax.axis_index("s") * 2 + lax.axis_index("c")
    lane = lax.iota(jnp.int32, 16)
    slabs = (slab0, slab1)
    outs = (out0, out1)
    sins = (sin0, sin1)
    souts = (sout0, sout1)

    def in_desc(j, b):
        c0 = pl.multiple_of(j * CW, CW)
        return pltpu.make_async_copy(embt_hbm.at[:, pl.ds(c0, CW)],
                                     slabs[b], sins[b])

    def out_desc(j, b):
        c0 = pl.multiple_of(j * CW, CW)
        return pltpu.make_async_copy(outs[b], embd_hbm.at[pl.ds(c0, CW)],
                                     souts[b])

    for b in range(2):
        j = wid + b * NW

        @pl.when(j < NFULL)
        def _():
            in_desc(j, b).start()

    @pl.loop(0, CPW, step=2)
    def _chunks(k):
        for b in range(2):
            j = wid + (k + b) * NW

            @pl.when(j < NFULL)
            def _():
                in_desc(j, b).wait()

                @pl.when(k + b >= 2)
                def _():
                    out_desc(j, b).wait()

                @plsc.parallel_loop(0, CW, unroll=8)
                def _row(r):
                    col = jnp.full((16,), r, jnp.int32)
                    for gi in range(D // 16):
                        v = plsc.load_gather(slabs[b], [lane + gi * 16, col])
                        outs[b][r, pl.ds(gi * 16, 16)] = v
                        outs[b][r, pl.ds(D + gi * 16, 16)] = v

                nj = j + 2 * NW

                @pl.when(nj < NFULL)
                def _():
                    in_desc(nj, b).start()

                out_desc(j, b).start()

    for b in range(2):
        last = wid + (CPW - 2 + b) * NW

        @pl.when(last < NFULL)
        def _():
            out_desc(last, b).wait()

    @pl.when(wid == 1)
    def _ragged():
        pltpu.sync_copy(tail_hbm, out0.at[pl.ds(0, RAG)])
        pltpu.sync_copy(out0.at[pl.ds(0, RAG)],
                        embd_hbm.at[pl.ds(RAG_C0, RAG)])


@functools.partial(
    pl.kernel,
    out_type=jax.ShapeDtypeStruct((B, 2 * D), jnp.float32),
    mesh=_mesh,
    scratch_types=[
        pltpu.VMEM((IPW,), jnp.int32),
        pltpu.VMEM((L, SPW), jnp.int32),
        pltpu.VMEM((SPW, 2 * D), jnp.float32),
        pltpu.SemaphoreType.DMA,
    ],
    compiler_params=pltpu.CompilerParams(needs_layout_passes=False),
)
def _sc_pool(x_hbm, embd_hbm, out_hbm, xb_v, idx_v, acc_v, sem):
    wid = lax.axis_index("s") * 2 + lax.axis_index("c")
    pltpu.sync_copy(x_hbm.at[pl.ds(wid * IPW, IPW)], xb_v)

    lane = lax.iota(jnp.int32, 16)

    @plsc.parallel_loop(0, L, unroll=8)
    def _tr(r):
        base = lane * L + r
        for gi in range(SPW // 16):
            v = plsc.load_gather(xb_v, [base + (gi * 16 * L)])
            idx_v[r, pl.ds(gi * 16, 16)] = v

    pltpu.sync_copy(embd_hbm.at[idx_v.at[0]], acc_v)

    @pl.loop(0, L - 1)
    def _fire(i):
        pltpu.async_copy(embd_hbm.at[idx_v.at[i + 1]], acc_v, sem, add=True)

        @pl.when(i >= WINDOW - 1)
        def _():
            pltpu.make_async_copy(embd_hbm.at[idx_v.at[0]], acc_v, sem).wait()

    @pl.loop(0, WINDOW - 1)
    def _drain(_):
        pltpu.make_async_copy(embd_hbm.at[idx_v.at[0]], acc_v, sem).wait()

    pltpu.sync_copy(acc_v, out_hbm.at[pl.ds(wid * SPW, SPW)])


def _mlp_body(s_ref, w1_ref, b1_ref, w2_ref, b2_ref, o_ref):
    x = s_ref[...] * np.float32(1.0 / L)
    h = lax.dot_general(x, w1_ref[...], (((1,), (1,)), ((), ())),
                        preferred_element_type=jnp.float32)
    h = jnp.maximum(h + b1_ref[...], 0.0)
    o_ref[...] = lax.dot_general(h, w2_ref[...], (((1,), (1,)), ((), ())),
                                 preferred_element_type=jnp.float32) + b2_ref[...]


def _mlp(sums, W1, b1, W2, b2):
    return pl.pallas_call(
        _mlp_body,
        out_shape=jax.ShapeDtypeStruct((B, NCLS), jnp.float32),
    )(sums, W1, b1.reshape(1, HID), W2, b2.reshape(1, NCLS))


def kernel(x_in, emb, W1, b1, W2, b2):
    embt = jnp.transpose(emb)
    tail = emb[RAG_C0:]
    tail = jnp.concatenate([tail, tail], axis=1)
    embd = _sc_format(embt, tail)
    sums = _sc_pool(x_in.reshape(-1), embd)[:, :D]
    return _mlp(sums, W1, b1, W2, b2)


# speedup vs baseline: 1.6982x; 1.0038x over previous
import functools

import jax
import jax.numpy as jnp
import numpy as np
from jax import lax
from jax.experimental import pallas as pl
from jax.experimental.pallas import tpu as pltpu
from jax.experimental.pallas import tpu_sc as plsc

VOCAB = 1000000
D = 64
HID = 128
NCLS = 2
B = 4096
L = 200

NW = 32
SPW = B // NW
IPW = SPW * L
WINDOW = 16

CW = 256
NFULL = VOCAB // CW
RAG = VOCAB - NFULL * CW
RAG_C0 = NFULL * CW
CPW = (NFULL + NW - 1) // NW

_mesh = plsc.VectorSubcoreMesh(core_axis_name="c", subcore_axis_name="s")


@functools.partial(
    pl.kernel,
    out_type=jax.ShapeDtypeStruct((VOCAB, 2 * D), jnp.float32),
    mesh=_mesh,
    scratch_types=[
        pltpu.VMEM((D, CW), jnp.float32),
        pltpu.VMEM((D, CW), jnp.float32),
        pltpu.VMEM((CW, 2 * D), jnp.float32),
        pltpu.VMEM((CW, 2 * D), jnp.float32),
        pltpu.SemaphoreType.DMA,
        pltpu.SemaphoreType.DMA,
        pltpu.SemaphoreType.DMA,
        pltpu.SemaphoreType.DMA,
    ],
    compiler_params=pltpu.CompilerParams(needs_layout_passes=False),
)
def _sc_format(embt_hbm, tail_hbm, embd_hbm, slab0, slab1, out0, out1,
               sin0, sin1, sout0, sout1):
    wid = lax.axis_index("s") * 2 + lax.axis_index("c")
    lane = lax.iota(jnp.int32, 16)
    slabs = (slab0, slab1)
    outs = (out0, out1)
    sins = (sin0, sin1)
    souts = (sout0, sout1)

    def in_desc(j, b):
        c0 = pl.multiple_of(j * CW, CW)
        return pltpu.make_async_copy(embt_hbm.at[:, pl.ds(c0, CW)],
                                     slabs[b], sins[b])

    def out_desc(j, b):
        c0 = pl.multiple_of(j * CW, CW)
        return pltpu.make_async_copy(outs[b], embd_hbm.at[pl.ds(c0, CW)],
                                     souts[b])

    for b in range(2):
        j = wid + b * NW

        @pl.when(j < NFULL)
        def _():
            in_desc(j, b).start()

    @pl.loop(0, CPW, step=2)
    def _chunks(k):
        for b in range(2):
            j = wid + (k + b) * NW

            @pl.when(j < NFULL)
            def _():
                in_desc(j, b).wait()

                @pl.when(k + b >= 2)
                def _():
                    out_desc(j, b).wait()

                @plsc.parallel_loop(0, CW, unroll=16)
                def _row(r):
                    col = jnp.full((16,), r, jnp.int32)
                    for gi in range(D // 16):
                        v = plsc.load_gather(slabs[b], [lane + gi * 16, col])
                        outs[b][r, pl.ds(gi * 16, 16)] = v
                        outs[b][r, pl.ds(D + gi * 16, 16)] = v

                nj = j + 2 * NW

                @pl.when(nj < NFULL)
                def _():
                    in_desc(nj, b).start()

                out_desc(j, b).start()

    for b in range(2):
        last = wid + (CPW - 2 + b) * NW

        @pl.when(last < NFULL)
        def _():
            out_desc(last, b).wait()

    @pl.when(wid == 1)
    def _ragged():
        pltpu.sync_copy(tail_hbm, out0.at[pl.ds(0, RAG)])
        pltpu.sync_copy(out0.at[pl.ds(0, RAG)],
                        embd_hbm.at[pl.ds(RAG_C0, RAG)])


@functools.partial(
    pl.kernel,
    out_type=jax.ShapeDtypeStruct((B, 2 * D), jnp.float32),
    mesh=_mesh,
    scratch_types=[
        pltpu.VMEM((IPW,), jnp.int32),
        pltpu.VMEM((L, SPW), jnp.int32),
        pltpu.VMEM((SPW, 2 * D), jnp.float32),
        pltpu.SemaphoreType.DMA,
    ],
    compiler_params=pltpu.CompilerParams(needs_layout_passes=False),
)
def _sc_pool(x_hbm, embd_hbm, out_hbm, xb_v, idx_v, acc_v, sem):
    wid = lax.axis_index("s") * 2 + lax.axis_index("c")
    pltpu.sync_copy(x_hbm.at[pl.ds(wid * IPW, IPW)], xb_v)

    lane = lax.iota(jnp.int32, 16)

    @plsc.parallel_loop(0, L, unroll=8)
    def _tr(r):
        base = lane * L + r
        for gi in range(SPW // 16):
            v = plsc.load_gather(xb_v, [base + (gi * 16 * L)])
            idx_v[r, pl.ds(gi * 16, 16)] = v

    pltpu.sync_copy(embd_hbm.at[idx_v.at[0]], acc_v)

    @pl.loop(0, L - 1)
    def _fire(i):
        pltpu.async_copy(embd_hbm.at[idx_v.at[i + 1]], acc_v, sem, add=True)

        @pl.when(i >= WINDOW - 1)
        def _():
            pltpu.make_async_copy(embd_hbm.at[idx_v.at[0]], acc_v, sem).wait()

    @pl.loop(0, WINDOW - 1)
    def _drain(_):
        pltpu.make_async_copy(embd_hbm.at[idx_v.at[0]], acc_v, sem).wait()

    pltpu.sync_copy(acc_v, out_hbm.at[pl.ds(wid * SPW, SPW)])


def _mlp_body(s_ref, w1_ref, b1_ref, w2_ref, b2_ref, o_ref):
    x = s_ref[...] * np.float32(1.0 / L)
    h = lax.dot_general(x, w1_ref[...], (((1,), (1,)), ((), ())),
                        preferred_element_type=jnp.float32)
    h = jnp.maximum(h + b1_ref[...], 0.0)
    o_ref[...] = lax.dot_general(h, w2_ref[...], (((1,), (1,)), ((), ())),
                                 preferred_element_type=jnp.float32) + b2_ref[...]


def _mlp(sums, W1, b1, W2, b2):
    return pl.pallas_call(
        _mlp_body,
        out_shape=jax.ShapeDtypeStruct((B, NCLS), jnp.float32),
    )(sums, W1, b1.reshape(1, HID), W2, b2.reshape(1, NCLS))


def kernel(x_in, emb, W1, b1, W2, b2):
    embt = jnp.transpose(emb)
    tail = emb[RAG_C0:]
    tail = jnp.concatenate([tail, tail], axis=1)
    embd = _sc_format(embt, tail)
    sums = _sc_pool(x_in.reshape(-1), embd)[:, :D]
    return _mlp(sums, W1, b1, W2, b2)


# speedup vs baseline: 2.5429x; 1.4974x over previous
import functools

import jax
import jax.numpy as jnp
import numpy as np
from jax import lax
from jax.experimental import pallas as pl
from jax.experimental.pallas import tpu as pltpu
from jax.experimental.pallas import tpu_sc as plsc

VOCAB = 1000000
D = 64
HID = 128
NCLS = 2
B = 4096
L = 200

NW = 32
SPW = B // NW
WINDOW = 16

_mesh = plsc.VectorSubcoreMesh(core_axis_name="c", subcore_axis_name="s")


@functools.partial(
    pl.kernel,
    out_type=jax.ShapeDtypeStruct((B, D), jnp.float32),
    mesh=_mesh,
    scratch_types=[
        pltpu.VMEM((L, SPW), jnp.int32),
        pltpu.VMEM((SPW, D), jnp.float32),
        pltpu.SemaphoreType.DMA,
    ],
    compiler_params=pltpu.CompilerParams(use_tc_tiling_on_sc=False,
                                         needs_layout_passes=False),
)
def _sc_pool(xt_hbm, emb_hbm, out_hbm, idx_v, acc_v, sem):
    wid = lax.axis_index("s") * 2 + lax.axis_index("c")
    col = wid * SPW
    pltpu.sync_copy(xt_hbm.at[:, pl.ds(col, SPW)], idx_v)

    pltpu.sync_copy(emb_hbm.at[idx_v.at[0]], acc_v)

    @pl.loop(0, L - 1)
    def _fire(i):
        pltpu.async_copy(emb_hbm.at[idx_v.at[i + 1]], acc_v, sem, add=True)

        @pl.when(i >= WINDOW - 1)
        def _():
            pltpu.make_async_copy(emb_hbm.at[idx_v.at[0]], acc_v, sem).wait()

    @pl.loop(0, WINDOW - 1)
    def _drain(_):
        pltpu.make_async_copy(emb_hbm.at[idx_v.at[0]], acc_v, sem).wait()

    pltpu.sync_copy(acc_v, out_hbm.at[pl.ds(wid * SPW, SPW)])


def _mlp_body(s_ref, w1_ref, b1_ref, w2_ref, b2_ref, o_ref):
    x = s_ref[...] * np.float32(1.0 / L)
    h = lax.dot_general(x, w1_ref[...], (((1,), (1,)), ((), ())),
                        preferred_element_type=jnp.float32)
    h = jnp.maximum(h + b1_ref[...], 0.0)
    o_ref[...] = lax.dot_general(h, w2_ref[...], (((1,), (1,)), ((), ())),
                                 preferred_element_type=jnp.float32) + b2_ref[...]


def _mlp(sums, W1, b1, W2, b2):
    return pl.pallas_call(
        _mlp_body,
        out_shape=jax.ShapeDtypeStruct((B, NCLS), jnp.float32),
    )(sums, W1, b1.reshape(1, HID), W2, b2.reshape(1, NCLS))


def kernel(x_in, emb, W1, b1, W2, b2):
    x_t = jnp.transpose(x_in)
    sums = _sc_pool(x_t, emb)
    return _mlp(sums, W1, b1, W2, b2)
